# Initial kernel scaffold; baseline (speedup 1.0000x reference)
import jax
import jax.numpy as jnp
from jax.experimental import pallas as pl


def kernel(baselines, frames, gains):
    raise NotImplementedError("write your pallas kernel here")


# speedup vs baseline: 270.1941x; 270.1941x over previous
import functools

import jax
import jax.numpy as jnp
from jax import lax
from jax.experimental import pallas as pl
from jax.experimental.pallas import tpu as pltpu
from jax.experimental.pallas import tpu_sc as plsc

NSITES = 64
NTIMES = 8192
NBASE = 2016
NFRAMES = 4096

NC = 2
NS = 16
NW = NC * NS
L = 16

FPW = NFRAMES // NW
CHUNK = 4
NCHUNKS = FPW // CHUNK
ROWW = 2 * NBASE
GROUPS = NBASE // L


def _tc_clip(gt_ref, out_ref):
    x = gt_ref[...]
    x = (x + jnp.pi) % (2.0 * jnp.pi) - jnp.pi
    out_ref[...] = jnp.clip(x, -jnp.pi, jnp.pi)


def _sc_body(bl_hbm, fr1_hbm, fr2_hbm, cgt_hbm, gi_hbm, gj_hbm,
             idx1_v, idx2_v, cols_v, rows_v, gi_v, gj_v,
             sem_cols, sem_rows0, sem_rows1, sem_out0, sem_out1):
    wid = lax.axis_index("s") * NC + lax.axis_index("c")
    fbase = wid * FPW

    pltpu.sync_copy(fr1_hbm.at[pl.ds(fbase, FPW)], idx1_v)
    cols_cp = pltpu.async_copy(cgt_hbm.at[idx1_v], cols_v, sem_cols)
    pltpu.sync_copy(fr2_hbm.at[pl.ds(wid * NCHUNKS, NCHUNKS)], idx2_v)

    row_sems = (sem_rows0, sem_rows1)
    out_sems = (sem_out0, sem_out1)

    def start_rows(c):
        buf = c % 2
        return pltpu.async_copy(
            bl_hbm.at[idx2_v.at[c]], rows_v.at[buf], row_sems[buf])

    rows_pending = [start_rows(0), None]
    cols_cp.wait()

    iota2 = lax.iota(jnp.int32, L) * 2
    pending_out = [None, None]

    for c in range(NCHUNKS):
        buf = c % 2
        if c + 1 < NCHUNKS:
            rows_pending[1 - buf] = start_rows(c + 1)
        rows_pending[buf].wait()
        if pending_out[buf] is not None:
            pending_out[buf][0].wait()
            pending_out[buf][1].wait()

        buf_vec = jnp.full((L,), buf, jnp.int32)
        for fc in range(CHUNK):
            fw = c * CHUNK + fc
            fw_vec = jnp.full((L,), fw, jnp.int32)
            fc_vec = jnp.full((L,), fc, jnp.int32)

            def body(g, _, buf_vec=buf_vec, fc_vec=fc_vec, fw_vec=fw_vec,
                     buf=buf, fc=fc):
                e = iota2 + g * 32
                ivals = plsc.load_gather(rows_v, [buf_vec, fc_vec, e])
                jvals = plsc.load_gather(rows_v, [buf_vec, fc_vec, e + 1])
                gi = plsc.load_gather(cols_v, [fw_vec, ivals])
                gj = plsc.load_gather(cols_v, [fw_vec, jvals])
                gi_v[buf, fc, pl.ds(g * L, L)] = gi
                gj_v[buf, fc, pl.ds(g * L, L)] = gj
                return 0

            lax.fori_loop(0, GROUPS, body, 0)

        out_row = pl.ds(fbase + c * CHUNK, CHUNK)
        cp_gi = pltpu.async_copy(gi_v.at[buf], gi_hbm.at[out_row], out_sems[buf])
        cp_gj = pltpu.async_copy(gj_v.at[buf], gj_hbm.at[out_row], out_sems[buf])
        pending_out[buf] = (cp_gi, cp_gj)

    for p in pending_out:
        if p is not None:
            p[0].wait()
            p[1].wait()


_sc_call = pl.kernel(
    _sc_body,
    out_type=(
        jax.ShapeDtypeStruct((NFRAMES, NBASE), jnp.float32),
        jax.ShapeDtypeStruct((NFRAMES, NBASE), jnp.float32),
    ),
    mesh=plsc.VectorSubcoreMesh(
        core_axis_name="c", subcore_axis_name="s",
        num_cores=NC, num_subcores=NS),
    compiler_params=pltpu.CompilerParams(
        needs_layout_passes=False, use_tc_tiling_on_sc=False),
    scratch_types=[
        pltpu.VMEM((FPW,), jnp.int32),
        pltpu.VMEM((NCHUNKS, CHUNK), jnp.int32),
        pltpu.VMEM((FPW, NSITES), jnp.float32),
        pltpu.VMEM((2, CHUNK, ROWW), jnp.int32),
        pltpu.VMEM((2, CHUNK, NBASE), jnp.float32),
        pltpu.VMEM((2, CHUNK, NBASE), jnp.float32),
        pltpu.SemaphoreType.DMA,
        pltpu.SemaphoreType.DMA,
        pltpu.SemaphoreType.DMA,
        pltpu.SemaphoreType.DMA,
        pltpu.SemaphoreType.DMA,
    ],
)


def kernel(baselines, frames, gains):
    bl_flat = baselines.reshape(NTIMES, ROWW)
    fr2 = frames.reshape(NFRAMES // CHUNK, CHUNK)
    gt = gains.T
    cgt = pl.pallas_call(
        _tc_clip,
        out_shape=jax.ShapeDtypeStruct((NTIMES, NSITES), jnp.float32),
    )(gt)
    gi, gj = _sc_call(bl_flat, frames, fr2, cgt)
    return gi, gj
